# baseline (speedup 1.0000x reference)
import jax
import jax.numpy as jnp
from jax import lax
from jax.experimental import pallas as pl
from jax.experimental.pallas import tpu as pltpu
from jax.experimental.pallas import tpu_sc as plsc

S, H, E = 2048, 1024, 16
BLK = 128
P = S * 2 + E * BLK
NBLKMAX = P // BLK
CHUNK = 512
NCH = (2 * S) // CHUNK
NW = 32
TPW = S // NW


def _f32(x):
    return x.astype(jnp.float32)


def _gate_route_body(tokens_ref, gateW_ref, pos_ref, w01_ref, meta_ref):
    x = tokens_ref[...]
    logits = lax.dot_general(x, gateW_ref[...], (((1,), (1,)), ((), ())),
                             preferred_element_type=jnp.float32)
    w = jax.nn.softmax(logits, axis=-1)
    lane = lax.broadcasted_iota(jnp.int32, (S, E), 1)
    m0 = jnp.max(w, axis=-1, keepdims=True)
    i0 = jnp.min(jnp.where(w == m0, lane, E), axis=-1, keepdims=True)
    wm = jnp.where(lane == i0, -1.0, w)
    m1 = jnp.max(wm, axis=-1, keepdims=True)
    i1 = jnp.min(jnp.where(wm == m1, lane, E), axis=-1, keepdims=True)
    oh0 = _f32(lane == i0)
    oh1 = _f32(lane == i1)

    r512 = lax.broadcasted_iota(jnp.int32, (CHUNK, CHUNK), 0)
    c512 = lax.broadcasted_iota(jnp.int32, (CHUNK, CHUNK), 1)
    ident = _f32(r512 == c512)
    upper = _f32(r512 < c512)
    re16 = lax.broadcasted_iota(jnp.int32, (E, E), 0)
    ce16 = lax.broadcasted_iota(jnp.int32, (E, E), 1)
    m16 = _f32(ce16 < re16)

    def transpose(mat):
        return lax.dot_general(mat, ident, (((0,), (0,)), ((), ())),
                               preferred_element_type=jnp.float32)

    ec = jnp.concatenate([oh0, oh1], axis=0)
    running = jnp.zeros((E, 1), jnp.float32)
    ecT, rankT, excl = [], [], []
    for c in range(NCH):
        ec_c = ec[c * CHUNK:(c + 1) * CHUNK]
        t = transpose(ec_c)
        rk = lax.dot_general(t, upper, (((1,), (0,)), ((), ())),
                             preferred_element_type=jnp.float32)
        ecT.append(t)
        rankT.append(rk)
        excl.append(running)
        running = running + jnp.sum(t, axis=1, keepdims=True)

    counts = running
    padded = jnp.floor((counts + (BLK - 1)) * (1.0 / BLK)) * BLK
    pad_off = lax.dot_general(m16, padded, (((1,), (0,)), ((), ())),
                              preferred_element_type=jnp.float32)

    pos_chunks, w_chunks = [], []
    for c in range(NCH):
        base = rankT[c] + excl[c] + pad_off
        pos_c = jnp.sum(ecT[c] * base, axis=0, keepdims=True)
        pos_chunks.append(pos_c)
    for c in range(NCH):
        src = m0 if c < NCH // 2 else m1
        m_c = src[(c % (NCH // 2)) * CHUNK:((c % (NCH // 2)) + 1) * CHUNK]
        w_chunks.append(transpose(m_c))

    pos0 = jnp.concatenate(pos_chunks[:NCH // 2], axis=1)
    pos1 = jnp.concatenate(pos_chunks[NCH // 2:], axis=1)
    pos_ref[...] = jnp.concatenate([pos0, pos1], axis=0).astype(jnp.int32)
    w0 = jnp.concatenate(w_chunks[:NCH // 2], axis=1)
    w1 = jnp.concatenate(w_chunks[NCH // 2:], axis=1)
    w01_ref[...] = jnp.concatenate([w0, w1], axis=0)

    total = jnp.sum(padded)
    nblk = total * (1.0 / BLK)
    biota = lax.broadcasted_iota(jnp.int32, (1, 64), 1)
    thresh = _f32(biota) * float(BLK)
    eid = jnp.sum(_f32(pad_off <= thresh), axis=0, keepdims=True) - 1.0
    meta = jnp.where(biota == 48, nblk, eid)
    meta_ref[...] = meta.astype(jnp.int32)


def _gate_route(flat_tokens, gate_W):
    return pl.pallas_call(
        _gate_route_body,
        out_shape=(
            jax.ShapeDtypeStruct((2, S), jnp.int32),
            jax.ShapeDtypeStruct((2, S), jnp.float32),
            jax.ShapeDtypeStruct((1, 64), jnp.int32),
        ),
    )(flat_tokens, gate_W)


def _dispatch(flat_tokens, pos, w01):
    mesh = plsc.VectorSubcoreMesh(core_axis_name="c", subcore_axis_name="s")

    def body(tokens_hbm, pos_hbm, w01_hbm, sx_hbm, sw_hbm,
             idx0_v, idx1_v, rows_v, w0_v, w1_v, sem):
        wid = lax.axis_index("s") * 2 + lax.axis_index("c")
        base = wid * TPW
        pltpu.sync_copy(pos_hbm.at[0, pl.ds(base, TPW)], idx0_v)
        pltpu.sync_copy(pos_hbm.at[1, pl.ds(base, TPW)], idx1_v)
        pltpu.sync_copy(w01_hbm.at[0, pl.ds(base, TPW)], w0_v)
        pltpu.sync_copy(w01_hbm.at[1, pl.ds(base, TPW)], w1_v)
        pltpu.sync_copy(tokens_hbm.at[pl.ds(base, TPW)], rows_v)
        c0 = pltpu.async_copy(rows_v, sx_hbm.at[idx0_v], sem)
        c1 = pltpu.async_copy(rows_v, sx_hbm.at[idx1_v], sem)
        c2 = pltpu.async_copy(w0_v, sw_hbm.at[idx0_v], sem)
        c3 = pltpu.async_copy(w1_v, sw_hbm.at[idx1_v], sem)
        c0.wait(); c1.wait(); c2.wait(); c3.wait()

    return pl.kernel(
        body,
        out_type=(
            jax.ShapeDtypeStruct((P, H), jnp.float32),
            jax.ShapeDtypeStruct((P,), jnp.float32),
        ),
        mesh=mesh,
        scratch_types=[
            pltpu.VMEM((TPW,), jnp.int32),
            pltpu.VMEM((TPW,), jnp.int32),
            pltpu.VMEM((TPW, H), jnp.float32),
            pltpu.VMEM((TPW,), jnp.float32),
            pltpu.VMEM((TPW,), jnp.float32),
            pltpu.SemaphoreType.DMA,
        ],
    )(flat_tokens, pos, w01)


def _expert_body(eid_ref, nblk_ref, x_ref, W_hbm, b_ref, sw_ref, y_ref,
                 Wbuf_ref, sem_ref):
    j = pl.program_id(0)
    par = lax.fori_loop(
        1, j + 1,
        lambda i, p: p + (eid_ref[i] != eid_ref[i - 1]).astype(jnp.int32),
        jnp.int32(0)) & 1
    jm1 = jnp.maximum(j - 1, 0)
    changed = jnp.logical_or(j == 0, eid_ref[j] != eid_ref[jm1])

    @pl.when(j == 0)
    def _first_fetch():
        pltpu.make_async_copy(
            W_hbm.at[pl.ds(eid_ref[0], 1)], Wbuf_ref.at[pl.ds(0, 1)],
            sem_ref.at[0]).start()

    @pl.when(jnp.logical_and(j + 1 < NBLKMAX,
                             eid_ref[j + 1] != eid_ref[j]))
    def _prefetch_next():
        nxt = 1 - par
        pltpu.make_async_copy(
            W_hbm.at[pl.ds(eid_ref[j + 1], 1)], Wbuf_ref.at[pl.ds(nxt, 1)],
            sem_ref.at[nxt]).start()

    @pl.when(changed)
    def _wait_cur():
        pltpu.make_async_copy(
            W_hbm.at[pl.ds(eid_ref[j], 1)], Wbuf_ref.at[pl.ds(par, 1)],
            sem_ref.at[par]).wait()

    @pl.when(j < nblk_ref[0])
    def _():
        xw = lax.dot_general(
            x_ref[...].astype(jnp.bfloat16), Wbuf_ref[par],
            (((1,), (1,)), ((), ())), preferred_element_type=jnp.float32)
        y_ref[...] = jnp.maximum(xw + b_ref[0], 0.0) * sw_ref[...]


def _expert_matmul(sorted_x, sorted_w, expert_Wbf, expert_b, eid, nblk):
    return pl.pallas_call(
        _expert_body,
        grid_spec=pltpu.PrefetchScalarGridSpec(
            num_scalar_prefetch=2,
            grid=(NBLKMAX,),
            in_specs=[
                pl.BlockSpec((BLK, H), lambda j, eid, nblk: (j, 0)),
                pl.BlockSpec(memory_space=pl.ANY),
                pl.BlockSpec((1, 1, H), lambda j, eid, nblk: (eid[j], 0, 0)),
                pl.BlockSpec((BLK, 1), lambda j, eid, nblk: (j, 0)),
            ],
            out_specs=pl.BlockSpec((BLK, H), lambda j, eid, nblk: (j, 0)),
            scratch_shapes=[
                pltpu.VMEM((2, H, H), jnp.bfloat16),
                pltpu.SemaphoreType.DMA((2,)),
            ],
        ),
        out_shape=jax.ShapeDtypeStruct((P, H), jnp.float32),
        compiler_params=pltpu.CompilerParams(
            dimension_semantics=("arbitrary",)),
    )(eid, nblk, sorted_x, expert_Wbf, expert_b.reshape(E, 1, H),
      sorted_w.reshape(P, 1))


HALF = TPW // 2


def _combine(y, pos):
    mesh = plsc.VectorSubcoreMesh(core_axis_name="c", subcore_axis_name="s")

    def body(y_hbm, pos_hbm, out_hbm, idx0_v, idx1_v, y0_v, y1_v, sem):
        wid = lax.axis_index("s") * 2 + lax.axis_index("c")
        for half in range(2):
            base = wid * TPW + half * HALF
            pltpu.sync_copy(pos_hbm.at[0, pl.ds(base, HALF)], idx0_v)
            pltpu.sync_copy(pos_hbm.at[1, pl.ds(base, HALF)], idx1_v)
            c0 = pltpu.async_copy(y_hbm.at[idx0_v], y0_v, sem)
            c1 = pltpu.async_copy(y_hbm.at[idx1_v], y1_v, sem)
            c0.wait()
            c1.wait()

            def add_row(r, _):
                def add_vec(i, _):
                    sl = pl.ds(i * 16, 16)
                    y0_v[r, sl] += y1_v[r, sl]
                    return ()
                lax.fori_loop(0, H // 16, add_vec, (), unroll=8)
                return ()
            lax.fori_loop(0, HALF, add_row, ())
            pltpu.sync_copy(y0_v, out_hbm.at[pl.ds(base, HALF)])

    return pl.kernel(
        body,
        out_type=jax.ShapeDtypeStruct((S, H), jnp.float32),
        mesh=mesh,
        scratch_types=[
            pltpu.VMEM((HALF,), jnp.int32),
            pltpu.VMEM((HALF,), jnp.int32),
            pltpu.VMEM((HALF, H), jnp.float32),
            pltpu.VMEM((HALF, H), jnp.float32),
            pltpu.SemaphoreType.DMA,
        ],
    )(y, pos)


def kernel(tokens, gate_W, expert_W, expert_b):
    b, s, h = tokens.shape
    flat = tokens.reshape(s, h)
    expert_Wbf = expert_W.astype(jnp.bfloat16)
    pos, w01, meta = _gate_route(flat, gate_W)
    eid = meta[0, :NBLKMAX]
    nblk = meta[0, 48:49]
    sorted_x, sorted_w = _dispatch(flat, pos, w01)
    y = _expert_matmul(sorted_x, sorted_w, expert_Wbf, expert_b, eid, nblk)
    out = _combine(y, pos)
    return out.reshape(b, s, h)


# speedup vs baseline: 1.3387x; 1.3387x over previous
import jax
import jax.numpy as jnp
from jax import lax
from jax.experimental import pallas as pl
from jax.experimental.pallas import tpu as pltpu
from jax.experimental.pallas import tpu_sc as plsc

S, H, E = 2048, 1024, 16
BLK = 256
P = S * 2 + E * BLK
NBLKMAX = P // BLK
CHUNK = 512
NCH = (2 * S) // CHUNK
NW = 32
TPW = S // NW


def _f32(x):
    return x.astype(jnp.float32)


def _gate_route_body(tokens_ref, gateW_ref, pos_ref, w01_ref, meta_ref):
    x = tokens_ref[...]
    logits = lax.dot_general(x, gateW_ref[...], (((1,), (1,)), ((), ())),
                             preferred_element_type=jnp.float32)
    w = jax.nn.softmax(logits, axis=-1)
    lane = lax.broadcasted_iota(jnp.int32, (S, E), 1)
    m0 = jnp.max(w, axis=-1, keepdims=True)
    i0 = jnp.min(jnp.where(w == m0, lane, E), axis=-1, keepdims=True)
    wm = jnp.where(lane == i0, -1.0, w)
    m1 = jnp.max(wm, axis=-1, keepdims=True)
    i1 = jnp.min(jnp.where(wm == m1, lane, E), axis=-1, keepdims=True)
    oh0 = _f32(lane == i0)
    oh1 = _f32(lane == i1)

    r512 = lax.broadcasted_iota(jnp.int32, (CHUNK, CHUNK), 0)
    c512 = lax.broadcasted_iota(jnp.int32, (CHUNK, CHUNK), 1)
    ident = _f32(r512 == c512)
    upper = _f32(r512 < c512)
    re16 = lax.broadcasted_iota(jnp.int32, (E, E), 0)
    ce16 = lax.broadcasted_iota(jnp.int32, (E, E), 1)
    m16 = _f32(ce16 < re16)

    def transpose(mat):
        return lax.dot_general(mat, ident, (((0,), (0,)), ((), ())),
                               preferred_element_type=jnp.float32)

    ec = jnp.concatenate([oh0, oh1], axis=0)
    running = jnp.zeros((E, 1), jnp.float32)
    ecT, rankT, excl = [], [], []
    for c in range(NCH):
        ec_c = ec[c * CHUNK:(c + 1) * CHUNK]
        t = transpose(ec_c)
        rk = lax.dot_general(t, upper, (((1,), (0,)), ((), ())),
                             preferred_element_type=jnp.float32)
        ecT.append(t)
        rankT.append(rk)
        excl.append(running)
        running = running + jnp.sum(t, axis=1, keepdims=True)

    counts = running
    padded = jnp.floor((counts + (BLK - 1)) * (1.0 / BLK)) * BLK
    pad_off = lax.dot_general(m16, padded, (((1,), (0,)), ((), ())),
                              preferred_element_type=jnp.float32)

    pos_chunks, w_chunks = [], []
    for c in range(NCH):
        base = rankT[c] + excl[c] + pad_off
        pos_c = jnp.sum(ecT[c] * base, axis=0, keepdims=True)
        pos_chunks.append(pos_c)
    for c in range(NCH):
        src = m0 if c < NCH // 2 else m1
        m_c = src[(c % (NCH // 2)) * CHUNK:((c % (NCH // 2)) + 1) * CHUNK]
        w_chunks.append(transpose(m_c))

    pos0 = jnp.concatenate(pos_chunks[:NCH // 2], axis=1)
    pos1 = jnp.concatenate(pos_chunks[NCH // 2:], axis=1)
    pos_ref[...] = jnp.concatenate([pos0, pos1], axis=0).astype(jnp.int32)
    w0 = jnp.concatenate(w_chunks[:NCH // 2], axis=1)
    w1 = jnp.concatenate(w_chunks[NCH // 2:], axis=1)
    w01_ref[...] = jnp.concatenate([w0, w1], axis=0)

    total = jnp.sum(padded)
    nblk = total * (1.0 / BLK)
    biota = lax.broadcasted_iota(jnp.int32, (1, NBLKMAX), 1)
    thresh = _f32(biota) * float(BLK)
    eid = jnp.sum(_f32(pad_off <= thresh), axis=0, keepdims=True) - 1.0
    eid_prev = jnp.concatenate([eid[:, :1], eid[:, :-1]], axis=1)
    changes = _f32(eid != eid_prev)
    rb = lax.broadcasted_iota(jnp.int32, (NBLKMAX, NBLKMAX), 0)
    cb = lax.broadcasted_iota(jnp.int32, (NBLKMAX, NBLKMAX), 1)
    incl = _f32(rb <= cb)
    cum = lax.dot_general(changes, incl, (((1,), (0,)), ((), ())),
                          preferred_element_type=jnp.float32)
    par = cum - 2.0 * jnp.floor(cum * 0.5)
    nblk_seg = jnp.where(biota == 0, nblk, 0.0)
    meta = jnp.concatenate([eid, nblk_seg, par, jnp.zeros_like(eid)], axis=1)
    meta_ref[...] = meta.astype(jnp.int32)


def _gate_route(flat_tokens, gate_W):
    return pl.pallas_call(
        _gate_route_body,
        out_shape=(
            jax.ShapeDtypeStruct((2, S), jnp.int32),
            jax.ShapeDtypeStruct((2, S), jnp.float32),
            jax.ShapeDtypeStruct((1, 4 * NBLKMAX), jnp.int32),
        ),
    )(flat_tokens, gate_W)


def _dispatch(flat_tokens, pos, w01):
    mesh = plsc.VectorSubcoreMesh(core_axis_name="c", subcore_axis_name="s")

    def body(tokens_hbm, pos_hbm, w01_hbm, sx_hbm, sw_hbm,
             idx_v, rows_v, w_v, sem):
        wid = lax.axis_index("s") * 2 + lax.axis_index("c")
        base = wid * TPW
        pltpu.sync_copy(tokens_hbm.at[pl.ds(base, TPW)], rows_v)
        for slot in range(2):
            pltpu.sync_copy(pos_hbm.at[slot, pl.ds(base, TPW)], idx_v)
            pltpu.async_copy(rows_v, sx_hbm.at[idx_v], sem).wait()
            pltpu.sync_copy(w01_hbm.at[slot, pl.ds(base, TPW)], w_v)
            pltpu.async_copy(w_v, sw_hbm.at[idx_v], sem).wait()

    return pl.kernel(
        body,
        out_type=(
            jax.ShapeDtypeStruct((P, H), jnp.float32),
            jax.ShapeDtypeStruct((P,), jnp.float32),
        ),
        mesh=mesh,
        scratch_types=[
            pltpu.VMEM((TPW,), jnp.int32),
            pltpu.VMEM((TPW, H), jnp.float32),
            pltpu.VMEM((TPW,), jnp.float32),
            pltpu.SemaphoreType.DMA,
        ],
    )(flat_tokens, pos, w01)


def _expert_body(eid_ref, nblk_ref, slot_ref, x_ref, W_hbm, b_ref, sw_ref,
                 y_ref, Wbuf_ref, sem_ref):
    j = pl.program_id(0)
    par = slot_ref[j]
    jm1 = jnp.maximum(j - 1, 0)
    changed = jnp.logical_or(j == 0, eid_ref[j] != eid_ref[jm1])

    @pl.when(j == 0)
    def _first_fetch():
        pltpu.make_async_copy(
            W_hbm.at[pl.ds(eid_ref[0], 1)], Wbuf_ref.at[pl.ds(0, 1)],
            sem_ref.at[0]).start()

    @pl.when(jnp.logical_and(j + 1 < NBLKMAX,
                             eid_ref[j + 1] != eid_ref[j]))
    def _prefetch_next():
        nxt = 1 - par
        pltpu.make_async_copy(
            W_hbm.at[pl.ds(eid_ref[j + 1], 1)], Wbuf_ref.at[pl.ds(nxt, 1)],
            sem_ref.at[nxt]).start()

    @pl.when(changed)
    def _wait_cur():
        pltpu.make_async_copy(
            W_hbm.at[pl.ds(eid_ref[j], 1)], Wbuf_ref.at[pl.ds(par, 1)],
            sem_ref.at[par]).wait()

    @pl.when(j < nblk_ref[0])
    def _():
        xw = lax.dot_general(
            x_ref[...], Wbuf_ref[par],
            (((1,), (1,)), ((), ())), preferred_element_type=jnp.float32)
        y_ref[...] = jnp.maximum(xw + b_ref[0], 0.0) * sw_ref[...]


def _expert_matmul(sorted_x, sorted_w, expert_W, expert_b, eid, nblk, slot):
    return pl.pallas_call(
        _expert_body,
        grid_spec=pltpu.PrefetchScalarGridSpec(
            num_scalar_prefetch=3,
            grid=(NBLKMAX,),
            in_specs=[
                pl.BlockSpec((BLK, H), lambda j, eid, nblk, slot: (j, 0)),
                pl.BlockSpec(memory_space=pl.ANY),
                pl.BlockSpec((1, 1, H),
                             lambda j, eid, nblk, slot: (eid[j], 0, 0)),
                pl.BlockSpec((BLK, 1), lambda j, eid, nblk, slot: (j, 0)),
            ],
            out_specs=pl.BlockSpec((BLK, H),
                                   lambda j, eid, nblk, slot: (j, 0)),
            scratch_shapes=[
                pltpu.VMEM((2, H, H), jnp.float32),
                pltpu.SemaphoreType.DMA((2,)),
            ],
        ),
        out_shape=jax.ShapeDtypeStruct((P, H), jnp.float32),
        compiler_params=pltpu.CompilerParams(
            dimension_semantics=("arbitrary",)),
    )(eid, nblk, slot, sorted_x, expert_W, expert_b.reshape(E, 1, H),
      sorted_w.reshape(P, 1))


HALF = TPW // 2


def _combine(y, pos):
    mesh = plsc.VectorSubcoreMesh(core_axis_name="c", subcore_axis_name="s")

    def body(y_hbm, pos_hbm, out_hbm, idx_v, y0_v, y1_v, sem):
        wid = lax.axis_index("s") * 2 + lax.axis_index("c")
        for half in range(2):
            base = wid * TPW + half * HALF
            pltpu.sync_copy(pos_hbm.at[0, pl.ds(base, HALF)], idx_v)
            pltpu.async_copy(y_hbm.at[idx_v], y0_v, sem).wait()
            pltpu.sync_copy(pos_hbm.at[1, pl.ds(base, HALF)], idx_v)
            pltpu.async_copy(y_hbm.at[idx_v], y1_v, sem).wait()

            def add_row(r, _):
                def add_vec(i, _):
                    sl = pl.ds(i * 16, 16)
                    y0_v[r, sl] += y1_v[r, sl]
                    return ()
                lax.fori_loop(0, H // 16, add_vec, (), unroll=4)
                return ()
            lax.fori_loop(0, HALF, add_row, ())
            pltpu.sync_copy(y0_v, out_hbm.at[pl.ds(base, HALF)])

    return pl.kernel(
        body,
        out_type=jax.ShapeDtypeStruct((S, H), jnp.float32),
        mesh=mesh,
        scratch_types=[
            pltpu.VMEM((HALF,), jnp.int32),
            pltpu.VMEM((HALF, H), jnp.float32),
            pltpu.VMEM((HALF, H), jnp.float32),
            pltpu.SemaphoreType.DMA,
        ],
    )(y, pos)


def kernel(tokens, gate_W, expert_W, expert_b):
    b, s, h = tokens.shape
    flat = tokens.reshape(s, h)
    pos, w01, meta = _gate_route(flat, gate_W)
    eid = meta[0, :NBLKMAX]
    nblk = meta[0, NBLKMAX:NBLKMAX + 1]
    slot = meta[0, 2 * NBLKMAX:3 * NBLKMAX]
    sorted_x, sorted_w = _dispatch(flat, pos, w01)
    y = _expert_matmul(sorted_x, sorted_w, expert_W, expert_b, eid, nblk, slot)
    out = _combine(y, pos)
    return out.reshape(b, s, h)


# speedup vs baseline: 1.7260x; 1.2893x over previous
import jax
import jax.numpy as jnp
from jax import lax
from jax.experimental import pallas as pl
from jax.experimental.pallas import tpu as pltpu
from jax.experimental.pallas import tpu_sc as plsc

S, H, E = 2048, 1024, 16
BLK = 256
P = S * 2 + E * BLK
NBLKMAX = P // BLK
CHUNK = 512
NCH = (2 * S) // CHUNK
NW = 32
TPW = S // NW


def _f32(x):
    return x.astype(jnp.float32)


def _gate_route_body(tokens_ref, gateW_ref, pos_ref, w01_ref, meta_ref):
    x = tokens_ref[...]
    logits = lax.dot_general(x, gateW_ref[...], (((1,), (1,)), ((), ())),
                             preferred_element_type=jnp.float32)
    w = jax.nn.softmax(logits, axis=-1)
    lane = lax.broadcasted_iota(jnp.int32, (S, E), 1)
    m0 = jnp.max(w, axis=-1, keepdims=True)
    i0 = jnp.min(jnp.where(w == m0, lane, E), axis=-1, keepdims=True)
    wm = jnp.where(lane == i0, -1.0, w)
    m1 = jnp.max(wm, axis=-1, keepdims=True)
    i1 = jnp.min(jnp.where(wm == m1, lane, E), axis=-1, keepdims=True)
    oh0 = _f32(lane == i0)
    oh1 = _f32(lane == i1)

    r512 = lax.broadcasted_iota(jnp.int32, (CHUNK, CHUNK), 0)
    c512 = lax.broadcasted_iota(jnp.int32, (CHUNK, CHUNK), 1)
    ident = _f32(r512 == c512)
    upper = _f32(r512 < c512)
    re16 = lax.broadcasted_iota(jnp.int32, (E, E), 0)
    ce16 = lax.broadcasted_iota(jnp.int32, (E, E), 1)
    m16 = _f32(ce16 < re16)

    def transpose(mat):
        return lax.dot_general(mat, ident, (((0,), (0,)), ((), ())),
                               preferred_element_type=jnp.float32)

    ec = jnp.concatenate([oh0, oh1], axis=0)
    running = jnp.zeros((E, 1), jnp.float32)
    ecT, rankT, excl = [], [], []
    for c in range(NCH):
        ec_c = ec[c * CHUNK:(c + 1) * CHUNK]
        t = transpose(ec_c)
        rk = lax.dot_general(t, upper, (((1,), (0,)), ((), ())),
                             preferred_element_type=jnp.float32)
        ecT.append(t)
        rankT.append(rk)
        excl.append(running)
        running = running + jnp.sum(t, axis=1, keepdims=True)

    counts = running
    padded = jnp.floor((counts + (BLK - 1)) * (1.0 / BLK)) * BLK
    pad_off = lax.dot_general(m16, padded, (((1,), (0,)), ((), ())),
                              preferred_element_type=jnp.float32)

    pos_chunks = []
    for c in range(NCH):
        base = rankT[c] + excl[c] + pad_off
        pos_c = jnp.sum(ecT[c] * base, axis=0, keepdims=True)
        pos_chunks.append(pos_c)

    pos0 = jnp.concatenate(pos_chunks[:NCH // 2], axis=1)
    pos1 = jnp.concatenate(pos_chunks[NCH // 2:], axis=1)
    pos_ref[...] = jnp.concatenate([pos0, pos1], axis=0).astype(jnp.int32)
    w01_ref[0] = jnp.broadcast_to(m0, (S, E))
    w01_ref[1] = jnp.broadcast_to(m1, (S, E))

    total = jnp.sum(padded)
    nblk = total * (1.0 / BLK)
    biota = lax.broadcasted_iota(jnp.int32, (1, NBLKMAX), 1)
    thresh = _f32(biota) * float(BLK)
    eid = jnp.sum(_f32(pad_off <= thresh), axis=0, keepdims=True) - 1.0
    eid_prev = jnp.concatenate([eid[:, :1], eid[:, :-1]], axis=1)
    changes = _f32(eid != eid_prev)
    rb = lax.broadcasted_iota(jnp.int32, (NBLKMAX, NBLKMAX), 0)
    cb = lax.broadcasted_iota(jnp.int32, (NBLKMAX, NBLKMAX), 1)
    incl = _f32(rb <= cb)
    cum = lax.dot_general(changes, incl, (((1,), (0,)), ((), ())),
                          preferred_element_type=jnp.float32)
    par = cum - 2.0 * jnp.floor(cum * 0.5)
    nblk_seg = jnp.where(biota == 0, nblk, 0.0)
    meta = jnp.concatenate([eid, nblk_seg, par, jnp.zeros_like(eid)], axis=1)
    meta_ref[...] = meta.astype(jnp.int32)


def _gate_route(flat_tokens, gate_W):
    return pl.pallas_call(
        _gate_route_body,
        out_shape=(
            jax.ShapeDtypeStruct((2, S), jnp.int32),
            jax.ShapeDtypeStruct((2, S, E), jnp.float32),
            jax.ShapeDtypeStruct((1, 4 * NBLKMAX), jnp.int32),
        ),
    )(flat_tokens, gate_W)


def _dispatch(flat_tokens, pos):
    mesh = plsc.VectorSubcoreMesh(core_axis_name="c", subcore_axis_name="s")

    def body(tokens_hbm, pos_hbm, sx_hbm, idx0_v, idx1_v, rows_v, sem):
        wid = lax.axis_index("s") * 2 + lax.axis_index("c")
        base = wid * TPW
        pltpu.sync_copy(pos_hbm.at[0, pl.ds(base, TPW)], idx0_v)
        pltpu.sync_copy(pos_hbm.at[1, pl.ds(base, TPW)], idx1_v)
        pltpu.sync_copy(tokens_hbm.at[pl.ds(base, TPW)], rows_v)
        c0 = pltpu.async_copy(rows_v, sx_hbm.at[idx0_v], sem)
        c1 = pltpu.async_copy(rows_v, sx_hbm.at[idx1_v], sem)
        c0.wait()
        c1.wait()

    return pl.kernel(
        body,
        out_type=jax.ShapeDtypeStruct((P, H), jnp.float32),
        mesh=mesh,
        scratch_types=[
            pltpu.VMEM((TPW,), jnp.int32),
            pltpu.VMEM((TPW,), jnp.int32),
            pltpu.VMEM((TPW, H), jnp.float32),
            pltpu.SemaphoreType.DMA,
        ],
    )(flat_tokens, pos)


def _expert_body(eid_ref, nblk_ref, slot_ref, x_ref, W_hbm, b_ref,
                 y_ref, Wbuf_ref, Wbf_ref, sem_ref):
    j = pl.program_id(0)
    par = slot_ref[j]
    jm1 = jnp.maximum(j - 1, 0)
    changed = jnp.logical_or(j == 0, eid_ref[j] != eid_ref[jm1])

    @pl.when(j == 0)
    def _first_fetch():
        pltpu.make_async_copy(
            W_hbm.at[pl.ds(eid_ref[0], 1)], Wbuf_ref.at[pl.ds(0, 1)],
            sem_ref.at[0]).start()

    @pl.when(jnp.logical_and(j + 1 < NBLKMAX,
                             eid_ref[j + 1] != eid_ref[j]))
    def _prefetch_next():
        nxt = 1 - par
        pltpu.make_async_copy(
            W_hbm.at[pl.ds(eid_ref[j + 1], 1)], Wbuf_ref.at[pl.ds(nxt, 1)],
            sem_ref.at[nxt]).start()

    @pl.when(changed)
    def _wait_and_cast():
        pltpu.make_async_copy(
            W_hbm.at[pl.ds(eid_ref[j], 1)], Wbuf_ref.at[pl.ds(par, 1)],
            sem_ref.at[par]).wait()
        Wbf_ref[par] = Wbuf_ref[par].astype(jnp.bfloat16)

    @pl.when(j < nblk_ref[0])
    def _():
        xw = lax.dot_general(
            x_ref[...].astype(jnp.bfloat16), Wbf_ref[par],
            (((1,), (1,)), ((), ())), preferred_element_type=jnp.float32)
        y_ref[...] = jnp.maximum(xw + b_ref[0], 0.0)


def _expert_matmul(sorted_x, expert_W, expert_b, eid, nblk, slot):
    def clamp(j, nblk_ref):
        return jnp.minimum(j, nblk_ref[0] - 1)

    return pl.pallas_call(
        _expert_body,
        grid_spec=pltpu.PrefetchScalarGridSpec(
            num_scalar_prefetch=3,
            grid=(NBLKMAX,),
            in_specs=[
                pl.BlockSpec((BLK, H),
                             lambda j, eid, nblk, slot: (clamp(j, nblk), 0)),
                pl.BlockSpec(memory_space=pl.ANY),
                pl.BlockSpec((1, 1, H),
                             lambda j, eid, nblk, slot: (eid[j], 0, 0)),
            ],
            out_specs=pl.BlockSpec(
                (BLK, H), lambda j, eid, nblk, slot: (clamp(j, nblk), 0)),
            scratch_shapes=[
                pltpu.VMEM((2, H, H), jnp.float32),
                pltpu.VMEM((2, H, H), jnp.bfloat16),
                pltpu.SemaphoreType.DMA((2,)),
            ],
        ),
        out_shape=jax.ShapeDtypeStruct((P, H), jnp.float32),
        compiler_params=pltpu.CompilerParams(
            dimension_semantics=("arbitrary",)),
    )(eid, nblk, slot, sorted_x, expert_W, expert_b.reshape(E, 1, H))


HALF = TPW // 2


def _combine(y, pos, w01):
    mesh = plsc.VectorSubcoreMesh(core_axis_name="c", subcore_axis_name="s")

    def body(y_hbm, pos_hbm, w01_hbm, out_hbm,
             idx0_v, idx1_v, w0_v, w1_v, y0_v, y1_v, sem):
        wid = lax.axis_index("s") * 2 + lax.axis_index("c")
        for half in range(2):
            base = wid * TPW + half * HALF
            pltpu.sync_copy(pos_hbm.at[0, pl.ds(base, HALF)], idx0_v)
            pltpu.sync_copy(pos_hbm.at[1, pl.ds(base, HALF)], idx1_v)
            pltpu.sync_copy(w01_hbm.at[0, pl.ds(base, HALF)], w0_v)
            pltpu.sync_copy(w01_hbm.at[1, pl.ds(base, HALF)], w1_v)
            c0 = pltpu.async_copy(y_hbm.at[idx0_v], y0_v, sem)
            c1 = pltpu.async_copy(y_hbm.at[idx1_v], y1_v, sem)
            c0.wait()
            c1.wait()

            def comb_row(r, _):
                bc0 = w0_v[r, :]
                bc1 = w1_v[r, :]

                def comb_vec(i, _):
                    sl = pl.ds(i * 16, 16)
                    y0_v[r, sl] = bc0 * y0_v[r, sl] + bc1 * y1_v[r, sl]
                    return ()
                lax.fori_loop(0, H // 16, comb_vec, (), unroll=4)
                return ()
            lax.fori_loop(0, HALF, comb_row, ())
            pltpu.sync_copy(y0_v, out_hbm.at[pl.ds(base, HALF)])

    return pl.kernel(
        body,
        out_type=jax.ShapeDtypeStruct((S, H), jnp.float32),
        mesh=mesh,
        scratch_types=[
            pltpu.VMEM((HALF,), jnp.int32),
            pltpu.VMEM((HALF,), jnp.int32),
            pltpu.VMEM((HALF, E), jnp.float32),
            pltpu.VMEM((HALF, E), jnp.float32),
            pltpu.VMEM((HALF, H), jnp.float32),
            pltpu.VMEM((HALF, H), jnp.float32),
            pltpu.SemaphoreType.DMA,
        ],
    )(y, pos, w01)


def kernel(tokens, gate_W, expert_W, expert_b):
    b, s, h = tokens.shape
    flat = tokens.reshape(s, h)
    pos, w01, meta = _gate_route(flat, gate_W)
    eid = meta[0, :NBLKMAX]
    nblk = meta[0, NBLKMAX:NBLKMAX + 1]
    slot = meta[0, 2 * NBLKMAX:3 * NBLKMAX]
    sorted_x = _dispatch(flat, pos)
    y = _expert_matmul(sorted_x, expert_W, expert_b, eid, nblk, slot)
    out = _combine(y, pos, w01)
    return out.reshape(b, s, h)


# speedup vs baseline: 1.8946x; 1.0977x over previous
import jax
import jax.numpy as jnp
from jax import lax
from jax.experimental import pallas as pl
from jax.experimental.pallas import tpu as pltpu
from jax.experimental.pallas import tpu_sc as plsc

S, H, E = 2048, 1024, 16
BLK = 256
P = S * 2 + E * BLK
NBLKMAX = P // BLK
CHUNK = 512
NCH = (2 * S) // CHUNK
NW = 32
TPW = S // NW


def _f32(x):
    return x.astype(jnp.float32)


def _gate_route_body(tokens_ref, gateW_ref, pos_ref, w01_ref, meta_ref):
    x = tokens_ref[...]
    logits = lax.dot_general(x, gateW_ref[...], (((1,), (1,)), ((), ())),
                             preferred_element_type=jnp.float32)
    w = jax.nn.softmax(logits, axis=-1)
    lane = lax.broadcasted_iota(jnp.int32, (S, E), 1)
    m0 = jnp.max(w, axis=-1, keepdims=True)
    i0 = jnp.min(jnp.where(w == m0, lane, E), axis=-1, keepdims=True)
    wm = jnp.where(lane == i0, -1.0, w)
    m1 = jnp.max(wm, axis=-1, keepdims=True)
    i1 = jnp.min(jnp.where(wm == m1, lane, E), axis=-1, keepdims=True)
    oh0 = _f32(lane == i0)
    oh1 = _f32(lane == i1)

    r512 = lax.broadcasted_iota(jnp.int32, (CHUNK, CHUNK), 0)
    c512 = lax.broadcasted_iota(jnp.int32, (CHUNK, CHUNK), 1)
    ident = _f32(r512 == c512)
    upper = _f32(r512 < c512)
    re16 = lax.broadcasted_iota(jnp.int32, (E, E), 0)
    ce16 = lax.broadcasted_iota(jnp.int32, (E, E), 1)
    m16 = _f32(ce16 < re16)

    def transpose(mat):
        return lax.dot_general(mat, ident, (((0,), (0,)), ((), ())),
                               preferred_element_type=jnp.float32)

    ec = jnp.concatenate([oh0, oh1], axis=0)
    running = jnp.zeros((E, 1), jnp.float32)
    ecT, rankT, excl = [], [], []
    for c in range(NCH):
        ec_c = ec[c * CHUNK:(c + 1) * CHUNK]
        t = transpose(ec_c)
        rk = lax.dot_general(t, upper, (((1,), (0,)), ((), ())),
                             preferred_element_type=jnp.float32)
        ecT.append(t)
        rankT.append(rk)
        excl.append(running)
        running = running + jnp.sum(t, axis=1, keepdims=True)

    counts = running
    padded = jnp.floor((counts + (BLK - 1)) * (1.0 / BLK)) * BLK
    pad_off = lax.dot_general(m16, padded, (((1,), (0,)), ((), ())),
                              preferred_element_type=jnp.float32)

    pos_chunks = []
    for c in range(NCH):
        base = rankT[c] + excl[c] + pad_off
        pos_c = jnp.sum(ecT[c] * base, axis=0, keepdims=True)
        pos_chunks.append(pos_c)

    pos0 = jnp.concatenate(pos_chunks[:NCH // 2], axis=1)
    pos1 = jnp.concatenate(pos_chunks[NCH // 2:], axis=1)
    pos_ref[...] = jnp.concatenate([pos0, pos1], axis=0).astype(jnp.int32)
    w01_ref[0] = jnp.broadcast_to(m0, (S, E))
    w01_ref[1] = jnp.broadcast_to(m1, (S, E))

    total = jnp.sum(padded)
    nblk = total * (1.0 / BLK)
    biota = lax.broadcasted_iota(jnp.int32, (1, NBLKMAX), 1)
    thresh = _f32(biota) * float(BLK)
    eid = jnp.sum(_f32(pad_off <= thresh), axis=0, keepdims=True) - 1.0
    eid_prev = jnp.concatenate([eid[:, :1], eid[:, :-1]], axis=1)
    changes = _f32(eid != eid_prev)
    rb = lax.broadcasted_iota(jnp.int32, (NBLKMAX, NBLKMAX), 0)
    cb = lax.broadcasted_iota(jnp.int32, (NBLKMAX, NBLKMAX), 1)
    incl = _f32(rb <= cb)
    ident_b = _f32(rb == cb)
    run = lax.dot_general(changes, incl, (((1,), (0,)), ((), ())),
                          preferred_element_type=jnp.float32)
    ohlast = _f32(_f32(biota) == (nblk - 1.0))
    nruns = jnp.sum(run * ohlast) + 1.0
    first = jnp.maximum(changes, _f32(biota == 0))
    runT = lax.dot_general(ident_b, run, (((1,), (1,)), ((), ())),
                           preferred_element_type=jnp.float32)
    firstT = lax.dot_general(ident_b, first, (((1,), (1,)), ((), ())),
                             preferred_element_type=jnp.float32)
    runmat = _f32(runT == _f32(biota)) * firstT
    dist_eid = lax.dot_general(eid * first, runmat, (((1,), (0,)), ((), ())),
                               preferred_element_type=jnp.float32)
    seg1 = jnp.where(biota == 0, nblk, jnp.where(biota == 1, nruns, 0.0))
    meta = jnp.concatenate([eid, seg1, run, dist_eid], axis=1)
    meta_ref[...] = meta.astype(jnp.int32)


def _gate_route(flat_tokens, gate_W):
    return pl.pallas_call(
        _gate_route_body,
        out_shape=(
            jax.ShapeDtypeStruct((2, S), jnp.int32),
            jax.ShapeDtypeStruct((2, S, E), jnp.float32),
            jax.ShapeDtypeStruct((1, 4 * NBLKMAX), jnp.int32),
        ),
    )(flat_tokens, gate_W)


def _dispatch(flat_tokens, pos):
    mesh = plsc.VectorSubcoreMesh(core_axis_name="c", subcore_axis_name="s")

    def body(tokens_hbm, pos_hbm, sx_hbm, idx0_v, idx1_v, rows_v, sem):
        wid = lax.axis_index("s") * 2 + lax.axis_index("c")
        base = wid * TPW
        pltpu.sync_copy(pos_hbm.at[0, pl.ds(base, TPW)], idx0_v)
        pltpu.sync_copy(pos_hbm.at[1, pl.ds(base, TPW)], idx1_v)
        pltpu.sync_copy(tokens_hbm.at[pl.ds(base, TPW)], rows_v)
        c0 = pltpu.async_copy(rows_v, sx_hbm.at[idx0_v], sem)
        c1 = pltpu.async_copy(rows_v, sx_hbm.at[idx1_v], sem)
        c0.wait()
        c1.wait()

    return pl.kernel(
        body,
        out_type=jax.ShapeDtypeStruct((P, H), jnp.float32),
        mesh=mesh,
        scratch_types=[
            pltpu.VMEM((TPW,), jnp.int32),
            pltpu.VMEM((TPW,), jnp.int32),
            pltpu.VMEM((TPW, H), jnp.float32),
            pltpu.SemaphoreType.DMA,
        ],
    )(flat_tokens, pos)


def _expert_body(eid_ref, nblk_ref, nruns_ref, run_ref, dist_ref,
                 x_ref, W_hbm, b_ref, y_ref, Wbuf_ref, Wbf_ref, sem_ref):
    j = pl.program_id(0)
    run = run_ref[j]
    slot = lax.rem(run, 3)
    K = nruns_ref[0]
    jm1 = jnp.maximum(j - 1, 0)
    changed = jnp.logical_or(j == 0, eid_ref[j] != eid_ref[jm1])

    def fetch(k, s):
        pltpu.make_async_copy(
            W_hbm.at[pl.ds(dist_ref[k], 1)], Wbuf_ref.at[pl.ds(s, 1)],
            sem_ref.at[s]).start()

    @pl.when(j < nblk_ref[0])
    def _active():
        @pl.when(j == 0)
        def _prime():
            fetch(0, 0)

            @pl.when(K > 1)
            def _():
                fetch(1, 1)

            @pl.when(K > 2)
            def _():
                fetch(2, 2)

        @pl.when(jnp.logical_and(changed, j > 0))
        def _issue_ahead():
            @pl.when(run + 2 < K)
            def _():
                fetch(run + 2, lax.rem(run + 2, 3))

        @pl.when(changed)
        def _wait_and_cast():
            pltpu.make_async_copy(
                W_hbm.at[pl.ds(eid_ref[j], 1)], Wbuf_ref.at[pl.ds(slot, 1)],
                sem_ref.at[slot]).wait()
            Wbf_ref[slot] = Wbuf_ref[slot].astype(jnp.bfloat16)

        xw = lax.dot_general(
            x_ref[...].astype(jnp.bfloat16), Wbf_ref[slot],
            (((1,), (1,)), ((), ())), preferred_element_type=jnp.float32)
        y_ref[...] = jnp.maximum(xw + b_ref[0], 0.0)


def _expert_matmul(sorted_x, expert_W, expert_b, eid, nblk, nruns, run, dist):
    def clamp(j, nblk_ref):
        return jnp.minimum(j, nblk_ref[0] - 1)

    return pl.pallas_call(
        _expert_body,
        grid_spec=pltpu.PrefetchScalarGridSpec(
            num_scalar_prefetch=5,
            grid=(NBLKMAX,),
            in_specs=[
                pl.BlockSpec((BLK, H),
                             lambda j, e, n, k, r, d: (clamp(j, n), 0)),
                pl.BlockSpec(memory_space=pl.ANY),
                pl.BlockSpec((1, 1, H),
                             lambda j, e, n, k, r, d: (e[j], 0, 0)),
            ],
            out_specs=pl.BlockSpec(
                (BLK, H), lambda j, e, n, k, r, d: (clamp(j, n), 0)),
            scratch_shapes=[
                pltpu.VMEM((3, H, H), jnp.float32),
                pltpu.VMEM((3, H, H), jnp.bfloat16),
                pltpu.SemaphoreType.DMA((3,)),
            ],
        ),
        out_shape=jax.ShapeDtypeStruct((P, H), jnp.float32),
        compiler_params=pltpu.CompilerParams(
            dimension_semantics=("arbitrary",)),
    )(eid, nblk, nruns, run, dist, sorted_x, expert_W,
      expert_b.reshape(E, 1, H))


HALF = TPW // 2


def _combine(y, pos, w01):
    mesh = plsc.VectorSubcoreMesh(core_axis_name="c", subcore_axis_name="s")

    def body(y_hbm, pos_hbm, w01_hbm, out_hbm,
             idx0_v, idx1_v, w0_v, w1_v, y0_v, y1_v, sem):
        wid = lax.axis_index("s") * 2 + lax.axis_index("c")
        for half in range(2):
            base = wid * TPW + half * HALF
            pltpu.sync_copy(pos_hbm.at[0, pl.ds(base, HALF)], idx0_v)
            pltpu.sync_copy(pos_hbm.at[1, pl.ds(base, HALF)], idx1_v)
            pltpu.sync_copy(w01_hbm.at[0, pl.ds(base, HALF)], w0_v)
            pltpu.sync_copy(w01_hbm.at[1, pl.ds(base, HALF)], w1_v)
            c0 = pltpu.async_copy(y_hbm.at[idx0_v], y0_v, sem)
            c1 = pltpu.async_copy(y_hbm.at[idx1_v], y1_v, sem)
            c0.wait()
            c1.wait()

            def comb_row(r, _):
                bc0 = w0_v[r, :]
                bc1 = w1_v[r, :]

                def comb_vec(i, _):
                    sl = pl.ds(i * 16, 16)
                    y0_v[r, sl] = bc0 * y0_v[r, sl] + bc1 * y1_v[r, sl]
                    return ()
                lax.fori_loop(0, H // 16, comb_vec, (), unroll=4)
                return ()
            lax.fori_loop(0, HALF, comb_row, ())
            pltpu.sync_copy(y0_v, out_hbm.at[pl.ds(base, HALF)])

    return pl.kernel(
        body,
        out_type=jax.ShapeDtypeStruct((S, H), jnp.float32),
        mesh=mesh,
        scratch_types=[
            pltpu.VMEM((HALF,), jnp.int32),
            pltpu.VMEM((HALF,), jnp.int32),
            pltpu.VMEM((HALF, E), jnp.float32),
            pltpu.VMEM((HALF, E), jnp.float32),
            pltpu.VMEM((HALF, H), jnp.float32),
            pltpu.VMEM((HALF, H), jnp.float32),
            pltpu.SemaphoreType.DMA,
        ],
    )(y, pos, w01)


def kernel(tokens, gate_W, expert_W, expert_b):
    b, s, h = tokens.shape
    flat = tokens.reshape(s, h)
    pos, w01, meta = _gate_route(flat, gate_W)
    eid = meta[0, :NBLKMAX]
    nblk = meta[0, NBLKMAX:NBLKMAX + 1]
    nruns = meta[0, NBLKMAX + 1:NBLKMAX + 2]
    run = meta[0, 2 * NBLKMAX:3 * NBLKMAX]
    dist = meta[0, 3 * NBLKMAX:4 * NBLKMAX]
    sorted_x = _dispatch(flat, pos)
    y = _expert_matmul(sorted_x, expert_W, expert_b, eid, nblk, nruns,
                       run, dist)
    out = _combine(y, pos, w01)
    return out.reshape(b, s, h)


# speedup vs baseline: 2.0145x; 1.0633x over previous
import jax
import jax.numpy as jnp
from jax import lax
from jax.experimental import pallas as pl
from jax.experimental.pallas import tpu as pltpu
from jax.experimental.pallas import tpu_sc as plsc

S, H, E = 2048, 1024, 16
BLK = 256
P = S * 2 + E * BLK
NBLKMAX = P // BLK
CHUNK = 512
NCH = (2 * S) // CHUNK
NW = 32
TPW = S // NW


def _f32(x):
    return x.astype(jnp.float32)


def _gate_route_body(tokens_ref, gateW_ref, pos_ref, w01_ref, meta_ref):
    x = tokens_ref[...]
    logits = lax.dot_general(x, gateW_ref[...], (((1,), (1,)), ((), ())),
                             preferred_element_type=jnp.float32)
    w = jax.nn.softmax(logits, axis=-1)
    lane = lax.broadcasted_iota(jnp.int32, (S, E), 1)
    m0 = jnp.max(w, axis=-1, keepdims=True)
    i0 = jnp.min(jnp.where(w == m0, lane, E), axis=-1, keepdims=True)
    wm = jnp.where(lane == i0, -1.0, w)
    m1 = jnp.max(wm, axis=-1, keepdims=True)
    i1 = jnp.min(jnp.where(wm == m1, lane, E), axis=-1, keepdims=True)
    oh0 = _f32(lane == i0)
    oh1 = _f32(lane == i1)

    r512 = lax.broadcasted_iota(jnp.int32, (CHUNK, CHUNK), 0)
    c512 = lax.broadcasted_iota(jnp.int32, (CHUNK, CHUNK), 1)
    ident = _f32(r512 == c512)
    upper = _f32(r512 < c512)
    re16 = lax.broadcasted_iota(jnp.int32, (E, E), 0)
    ce16 = lax.broadcasted_iota(jnp.int32, (E, E), 1)
    m16 = _f32(ce16 < re16)

    def transpose(mat):
        return lax.dot_general(mat, ident, (((0,), (0,)), ((), ())),
                               preferred_element_type=jnp.float32)

    ec = jnp.concatenate([oh0, oh1], axis=0)
    running = jnp.zeros((E, 1), jnp.float32)
    ecT, rankT, excl = [], [], []
    for c in range(NCH):
        ec_c = ec[c * CHUNK:(c + 1) * CHUNK]
        t = transpose(ec_c)
        rk = lax.dot_general(t, upper, (((1,), (0,)), ((), ())),
                             preferred_element_type=jnp.float32)
        ecT.append(t)
        rankT.append(rk)
        excl.append(running)
        running = running + jnp.sum(t, axis=1, keepdims=True)

    counts = running
    padded = jnp.floor((counts + (BLK - 1)) * (1.0 / BLK)) * BLK
    pad_off = lax.dot_general(m16, padded, (((1,), (0,)), ((), ())),
                              preferred_element_type=jnp.float32)

    pos_chunks = []
    for c in range(NCH):
        base = rankT[c] + excl[c] + pad_off
        pos_c = jnp.sum(ecT[c] * base, axis=0, keepdims=True)
        pos_chunks.append(pos_c)

    pos0 = jnp.concatenate(pos_chunks[:NCH // 2], axis=1)
    pos1 = jnp.concatenate(pos_chunks[NCH // 2:], axis=1)
    pos_ref[...] = jnp.concatenate([pos0, pos1], axis=0).astype(jnp.int32)
    w01_ref[0] = jnp.broadcast_to(m0, (S, E))
    w01_ref[1] = jnp.broadcast_to(m1, (S, E))

    total = jnp.sum(padded)
    nblk = total * (1.0 / BLK)
    biota = lax.broadcasted_iota(jnp.int32, (1, NBLKMAX), 1)
    thresh = _f32(biota) * float(BLK)
    eid = jnp.sum(_f32(pad_off <= thresh), axis=0, keepdims=True) - 1.0
    eid_prev = jnp.concatenate([eid[:, :1], eid[:, :-1]], axis=1)
    changes = _f32(eid != eid_prev)
    rb = lax.broadcasted_iota(jnp.int32, (NBLKMAX, NBLKMAX), 0)
    cb = lax.broadcasted_iota(jnp.int32, (NBLKMAX, NBLKMAX), 1)
    incl = _f32(rb <= cb)
    ident_b = _f32(rb == cb)
    run = lax.dot_general(changes, incl, (((1,), (0,)), ((), ())),
                          preferred_element_type=jnp.float32)
    ohlast = _f32(_f32(biota) == (nblk - 1.0))
    nruns = jnp.sum(run * ohlast) + 1.0
    first = jnp.maximum(changes, _f32(biota == 0))
    runT = lax.dot_general(ident_b, run, (((1,), (1,)), ((), ())),
                           preferred_element_type=jnp.float32)
    firstT = lax.dot_general(ident_b, first, (((1,), (1,)), ((), ())),
                             preferred_element_type=jnp.float32)
    runmat = _f32(runT == _f32(biota)) * firstT
    dist_eid = lax.dot_general(eid * first, runmat, (((1,), (0,)), ((), ())),
                               preferred_element_type=jnp.float32)
    seg1 = jnp.where(biota == 0, nblk, jnp.where(biota == 1, nruns, 0.0))
    meta = jnp.concatenate([eid, seg1, run, dist_eid], axis=1)
    meta_ref[...] = meta.astype(jnp.int32)


def _gate_route(flat_tokens, gate_W):
    return pl.pallas_call(
        _gate_route_body,
        out_shape=(
            jax.ShapeDtypeStruct((2, S), jnp.int32),
            jax.ShapeDtypeStruct((2, S, E), jnp.float32),
            jax.ShapeDtypeStruct((1, 4 * NBLKMAX), jnp.int32),
        ),
    )(flat_tokens, gate_W)


def _dispatch(flat_tokens, pos):
    mesh = plsc.VectorSubcoreMesh(core_axis_name="c", subcore_axis_name="s")

    def body(tokens_hbm, pos_hbm, sx_hbm, idx0_v, idx1_v, rows_v, sem):
        wid = lax.axis_index("s") * 2 + lax.axis_index("c")
        base = wid * TPW
        pltpu.sync_copy(pos_hbm.at[0, pl.ds(base, TPW)], idx0_v)
        pltpu.sync_copy(pos_hbm.at[1, pl.ds(base, TPW)], idx1_v)
        pltpu.sync_copy(tokens_hbm.at[pl.ds(base, TPW)], rows_v)
        c0 = pltpu.async_copy(rows_v, sx_hbm.at[idx0_v], sem)
        c1 = pltpu.async_copy(rows_v, sx_hbm.at[idx1_v], sem)
        c0.wait()
        c1.wait()

    return pl.kernel(
        body,
        out_type=jax.ShapeDtypeStruct((P, H), jnp.float32),
        mesh=mesh,
        scratch_types=[
            pltpu.VMEM((TPW,), jnp.int32),
            pltpu.VMEM((TPW,), jnp.int32),
            pltpu.VMEM((TPW, H), jnp.float32),
            pltpu.SemaphoreType.DMA,
        ],
    )(flat_tokens, pos)


def _expert_body(eid_ref, nblk_ref, nruns_ref, run_ref, dist_ref,
                 x_ref, W_hbm, b_ref, y_ref, Wbuf_ref, Wbf_ref, sem_ref):
    j = pl.program_id(0)
    run = run_ref[j]
    slot = lax.rem(run, 3)
    K = nruns_ref[0]
    jm1 = jnp.maximum(j - 1, 0)
    changed = jnp.logical_or(j == 0, eid_ref[j] != eid_ref[jm1])

    def fetch(k, s):
        pltpu.make_async_copy(
            W_hbm.at[pl.ds(dist_ref[k], 1)], Wbuf_ref.at[pl.ds(s, 1)],
            sem_ref.at[s]).start()

    @pl.when(j < nblk_ref[0])
    def _active():
        @pl.when(j == 0)
        def _prime():
            fetch(0, 0)

            @pl.when(K > 1)
            def _():
                fetch(1, 1)

            @pl.when(K > 2)
            def _():
                fetch(2, 2)

        @pl.when(jnp.logical_and(changed, j > 0))
        def _issue_ahead():
            @pl.when(run + 2 < K)
            def _():
                fetch(run + 2, lax.rem(run + 2, 3))

        @pl.when(changed)
        def _wait_and_cast():
            pltpu.make_async_copy(
                W_hbm.at[pl.ds(eid_ref[j], 1)], Wbuf_ref.at[pl.ds(slot, 1)],
                sem_ref.at[slot]).wait()
            Wbf_ref[slot] = Wbuf_ref[slot].astype(jnp.bfloat16)

        xw = lax.dot_general(
            x_ref[...].astype(jnp.bfloat16), Wbf_ref[slot],
            (((1,), (1,)), ((), ())), preferred_element_type=jnp.float32)
        y_ref[...] = jnp.maximum(xw + b_ref[0], 0.0)


def _expert_matmul(sorted_x, expert_W, expert_b, eid, nblk, nruns, run, dist):
    def clamp(j, nblk_ref):
        return jnp.minimum(j, nblk_ref[0] - 1)

    return pl.pallas_call(
        _expert_body,
        grid_spec=pltpu.PrefetchScalarGridSpec(
            num_scalar_prefetch=5,
            grid=(NBLKMAX,),
            in_specs=[
                pl.BlockSpec((BLK, H),
                             lambda j, e, n, k, r, d: (clamp(j, n), 0)),
                pl.BlockSpec(memory_space=pl.ANY),
                pl.BlockSpec((1, 1, H),
                             lambda j, e, n, k, r, d: (e[j], 0, 0)),
            ],
            out_specs=pl.BlockSpec(
                (BLK, H), lambda j, e, n, k, r, d: (clamp(j, n), 0)),
            scratch_shapes=[
                pltpu.VMEM((3, H, H), jnp.float32),
                pltpu.VMEM((3, H, H), jnp.bfloat16),
                pltpu.SemaphoreType.DMA((3,)),
            ],
        ),
        out_shape=jax.ShapeDtypeStruct((P, H), jnp.float32),
        compiler_params=pltpu.CompilerParams(
            dimension_semantics=("arbitrary",)),
    )(eid, nblk, nruns, run, dist, sorted_x, expert_W,
      expert_b.reshape(E, 1, H))


HALF = TPW // 2


def _combine(y, pos, w01):
    mesh = plsc.VectorSubcoreMesh(core_axis_name="c", subcore_axis_name="s")

    NC4 = 4
    CH = TPW // NC4

    def body(y_hbm, pos_hbm, w01_hbm, out_hbm,
             idx0_v, idx1_v, w0_v, w1_v, ya0, ya1, yb0, yb1, sem):
        wid = lax.axis_index("s") * 2 + lax.axis_index("c")
        base = wid * TPW
        pltpu.sync_copy(pos_hbm.at[0, pl.ds(base, TPW)], idx0_v)
        pltpu.sync_copy(pos_hbm.at[1, pl.ds(base, TPW)], idx1_v)
        pltpu.sync_copy(w01_hbm.at[0, pl.ds(base, TPW)], w0_v)
        pltpu.sync_copy(w01_hbm.at[1, pl.ds(base, TPW)], w1_v)
        bufs = [(ya0, ya1), (yb0, yb1)]

        def gathers(c):
            b0, b1 = bufs[c % 2]
            g0 = pltpu.async_copy(
                y_hbm.at[idx0_v.at[pl.ds(c * CH, CH)]], b0, sem)
            g1 = pltpu.async_copy(
                y_hbm.at[idx1_v.at[pl.ds(c * CH, CH)]], b1, sem)
            return g0, g1

        pend = gathers(0)
        for c in range(NC4):
            pend[0].wait()
            pend[1].wait()
            if c + 1 < NC4:
                pend = gathers(c + 1)
            b0, b1 = bufs[c % 2]

            def comb_row(r, _):
                bc0 = w0_v[c * CH + r, :]
                bc1 = w1_v[c * CH + r, :]

                def comb_vec(i, _):
                    sl = pl.ds(i * 16, 16)
                    b0[r, sl] = bc0 * b0[r, sl] + bc1 * b1[r, sl]
                    return ()
                lax.fori_loop(0, H // 16, comb_vec, (), unroll=4)
                return ()
            lax.fori_loop(0, CH, comb_row, ())
            pltpu.sync_copy(b0, out_hbm.at[pl.ds(base + c * CH, CH)])

    return pl.kernel(
        body,
        out_type=jax.ShapeDtypeStruct((S, H), jnp.float32),
        mesh=mesh,
        scratch_types=[
            pltpu.VMEM((TPW,), jnp.int32),
            pltpu.VMEM((TPW,), jnp.int32),
            pltpu.VMEM((TPW, E), jnp.float32),
            pltpu.VMEM((TPW, E), jnp.float32),
            pltpu.VMEM((CH, H), jnp.float32),
            pltpu.VMEM((CH, H), jnp.float32),
            pltpu.VMEM((CH, H), jnp.float32),
            pltpu.VMEM((CH, H), jnp.float32),
            pltpu.SemaphoreType.DMA,
        ],
    )(y, pos, w01)


def kernel(tokens, gate_W, expert_W, expert_b):
    b, s, h = tokens.shape
    flat = tokens.reshape(s, h)
    pos, w01, meta = _gate_route(flat, gate_W)
    eid = meta[0, :NBLKMAX]
    nblk = meta[0, NBLKMAX:NBLKMAX + 1]
    nruns = meta[0, NBLKMAX + 1:NBLKMAX + 2]
    run = meta[0, 2 * NBLKMAX:3 * NBLKMAX]
    dist = meta[0, 3 * NBLKMAX:4 * NBLKMAX]
    sorted_x = _dispatch(flat, pos)
    y = _expert_matmul(sorted_x, expert_W, expert_b, eid, nblk, nruns,
                       run, dist)
    out = _combine(y, pos, w01)
    return out.reshape(b, s, h)
